# Initial kernel scaffold; baseline (speedup 1.0000x reference)
import jax
import jax.numpy as jnp
from jax.experimental import pallas as pl


def kernel(x, normal, W1, g1, b1, W2, g2, b2, W3, g3, b3, W4, g4, b4, W5, g5, b5, L1W, g6, b6, L2W, L2b, g7, b7, L3W, L3b):
    raise NotImplementedError("write your pallas kernel here")


# speedup vs baseline: 5.2419x; 5.2419x over previous
import functools

import jax
import jax.numpy as jnp
import numpy as np
from jax import lax
from jax.experimental import pallas as pl
from jax.experimental.pallas import tpu as pltpu

K = 20
N = 1024
_BN_C = float(1.0 / np.sqrt(1.0 + 1e-5))

_NT = (((1,), (1,)), ((), ()))
_NN = (((1,), (0,)), ((), ()))
_PH = lax.Precision.HIGHEST


def _leaky(y):
    return jnp.maximum(y, 0.2 * y)


def _edge_body(x_ref, wd_ref, wx_ref, sc_ref, sh_ref, o_ref, pd_ref, acc_ref):
    n = x_ref.shape[1]
    x = x_ref[0]
    xb = x.astype(jnp.bfloat16)
    g = lax.dot_general(xb, xb, _NT, preferred_element_type=jnp.float32)
    s = jnp.sum(x * x, axis=1)
    pd_ref[...] = 2.0 * g - s[:, None] - s[None, :]
    wdb = wd_ref[...].astype(jnp.bfloat16)
    t = lax.dot_general(xb, wx_ref[...].astype(jnp.bfloat16), _NT,
                        preferred_element_type=jnp.float32)
    acc_ref[...] = jnp.full(acc_ref.shape, -jnp.inf, jnp.float32)
    iota = lax.broadcasted_iota(jnp.int32, (n, n), 1)

    def step(_, carry):
        pd = pd_ref[...]
        m = jnp.max(pd, axis=1, keepdims=True)
        first = jnp.min(jnp.where(pd == m, iota, n), axis=1, keepdims=True)
        onehot = (iota == first).astype(jnp.float32)
        rows = lax.dot_general(onehot, x, _NN, precision=_PH)
        d = (rows - x).astype(jnp.bfloat16)
        hk = lax.dot_general(d, wdb, _NT, preferred_element_type=jnp.float32)
        acc_ref[...] = jnp.maximum(acc_ref[...], hk)
        pd_ref[...] = jnp.where(iota == first, -jnp.inf, pd)
        return carry

    lax.fori_loop(0, K, step, 0)
    o_ref[0] = _leaky((acc_ref[...] + t) * sc_ref[...][None, :] + sh_ref[...][None, :])


def _edge_layer(xb, W, gamma, beta):
    B, n, C = xb.shape
    O = W.shape[0]
    Wd = W[:, :C]
    Wx = W[:, C:]
    scale = gamma / np.sqrt(1.0 + 1e-5)
    return pl.pallas_call(
        _edge_body,
        grid=(B,),
        in_specs=[
            pl.BlockSpec((1, n, C), lambda b: (b, 0, 0)),
            pl.BlockSpec((O, C), lambda b: (0, 0)),
            pl.BlockSpec((O, C), lambda b: (0, 0)),
            pl.BlockSpec((O,), lambda b: (0,)),
            pl.BlockSpec((O,), lambda b: (0,)),
        ],
        out_specs=pl.BlockSpec((1, n, O), lambda b: (b, 0, 0)),
        out_shape=jax.ShapeDtypeStruct((B, n, O), jnp.float32),
        scratch_shapes=[
            pltpu.VMEM((n, n), jnp.float32),
            pltpu.VMEM((n, O), jnp.float32),
        ],
    )(xb, Wd, Wx, scale, beta)


def _head_body(x1_ref, x2_ref, x3_ref, x4_ref, wa_ref, wb_ref, wc_ref, wd_ref,
               g5_ref, b5_ref, l1a_ref, l1b_ref, g6_ref, b6_ref,
               l2_ref, l2b_ref, g7_ref, b7_ref, l3_ref, l3b_ref, o_ref):
    n = x1_ref.shape[1]
    bf = jnp.bfloat16
    f32 = jnp.float32
    h = (lax.dot_general(x1_ref[0].astype(bf), wa_ref[...].astype(bf), _NT, preferred_element_type=f32)
         + lax.dot_general(x2_ref[0].astype(bf), wb_ref[...].astype(bf), _NT, preferred_element_type=f32)
         + lax.dot_general(x3_ref[0].astype(bf), wc_ref[...].astype(bf), _NT, preferred_element_type=f32)
         + lax.dot_general(x4_ref[0].astype(bf), wd_ref[...].astype(bf), _NT, preferred_element_type=f32))
    h = _leaky(h * g5_ref[...][None, :] + b5_ref[...][None, :])
    p1 = jnp.max(h, axis=0)[None, :]
    p2 = (jnp.sum(h, axis=0) / n)[None, :]
    z = (lax.dot_general(p1.astype(bf), l1a_ref[...].astype(bf), _NT, preferred_element_type=f32)
         + lax.dot_general(p2.astype(bf), l1b_ref[...].astype(bf), _NT, preferred_element_type=f32))
    z = _leaky(z * g6_ref[...][None, :] + b6_ref[...][None, :])
    z = lax.dot_general(z.astype(bf), l2_ref[...].astype(bf), _NT, preferred_element_type=f32) + l2b_ref[...][None, :]
    z = _leaky(z * g7_ref[...][None, :] + b7_ref[...][None, :])
    z = lax.dot_general(z.astype(bf), l3_ref[...].astype(bf), _NT, preferred_element_type=f32) + l3b_ref[...][None, :]
    o_ref[0] = z


def _head(x1, x2, x3, x4, W5, g5, b5, L1W, g6, b6, L2W, L2b, g7, b7, L3W, L3b):
    B, n, _ = x1.shape
    emb = W5.shape[0]
    c1, c2, c3, c4 = x1.shape[2], x2.shape[2], x3.shape[2], x4.shape[2]
    Wa = W5[:, :c1]
    Wb = W5[:, c1:c1 + c2]
    Wc = W5[:, c1 + c2:c1 + c2 + c3]
    Wd = W5[:, c1 + c2 + c3:]
    L1a = L1W[:, :emb]
    L1b = L1W[:, emb:]
    full = lambda shape: pl.BlockSpec(shape, lambda b: (0,) * len(shape))
    return pl.pallas_call(
        _head_body,
        grid=(B,),
        in_specs=[
            pl.BlockSpec((1, n, c1), lambda b: (b, 0, 0)),
            pl.BlockSpec((1, n, c2), lambda b: (b, 0, 0)),
            pl.BlockSpec((1, n, c3), lambda b: (b, 0, 0)),
            pl.BlockSpec((1, n, c4), lambda b: (b, 0, 0)),
            full(Wa.shape), full(Wb.shape), full(Wc.shape), full(Wd.shape),
            full(g5.shape), full(b5.shape),
            full(L1a.shape), full(L1b.shape), full(g6.shape), full(b6.shape),
            full(L2W.shape), full(L2b.shape), full(g7.shape), full(b7.shape),
            full(L3W.shape), full(L3b.shape),
        ],
        out_specs=pl.BlockSpec((1, 1, 40), lambda b: (b, 0, 0)),
        out_shape=jax.ShapeDtypeStruct((B, 1, 40), jnp.float32),
    )(x1, x2, x3, x4, Wa, Wb, Wc, Wd, g5 / np.sqrt(1.0 + 1e-5), b5,
      L1a, L1b, g6 / np.sqrt(1.0 + 1e-5), b6, L2W, L2b, g7 / np.sqrt(1.0 + 1e-5), b7, L3W, L3b)


def kernel(x, normal, W1, g1, b1, W2, g2, b2, W3, g3, b3, W4, g4, b4, W5, g5, b5, L1W, g6, b6, L2W, L2b, g7, b7, L3W, L3b):
    del normal
    B = x.shape[0]
    x0 = jnp.concatenate(
        [x.astype(jnp.float32), jnp.zeros((B, N, 5), jnp.float32)], axis=2)
    W1p = jnp.concatenate(
        [W1[:, :3], jnp.zeros((64, 5), jnp.float32),
         W1[:, 3:], jnp.zeros((64, 5), jnp.float32)], axis=1)
    x1 = _edge_layer(x0, W1p, g1, b1)
    x2 = _edge_layer(x1, W2, g2, b2)
    x3 = _edge_layer(x2, W3, g3, b3)
    x4 = _edge_layer(x3, W4, g4, b4)
    out = _head(x1, x2, x3, x4, W5, g5, b5, L1W, g6, b6,
                L2W, L2b, g7, b7, L3W, L3b)
    return out.reshape(B, 40)


# speedup vs baseline: 8.5924x; 1.6392x over previous
import functools

import jax
import jax.numpy as jnp
import numpy as np
from jax import lax
from jax.experimental import pallas as pl
from jax.experimental.pallas import tpu as pltpu
from jax.experimental.pallas import tpu_sc as plsc

K = 20
N = 1024
_BN_C = float(1.0 / np.sqrt(1.0 + 1e-5))

_NT = (((1,), (1,)), ((), ()))
_NN = (((1,), (0,)), ((), ()))
_PH = lax.Precision.HIGHEST


def _leaky(y):
    return jnp.maximum(y, 0.2 * y)


def _edge_body(x_ref, wd_ref, wx_ref, sc_ref, sh_ref, o_ref, pd_ref, acc_ref):
    n = x_ref.shape[1]
    x = x_ref[0]
    xb = x.astype(jnp.bfloat16)
    g = lax.dot_general(xb, xb, _NT, preferred_element_type=jnp.float32)
    s = jnp.sum(x * x, axis=1)
    pd_ref[...] = 2.0 * g - s[:, None] - s[None, :]
    wdb = wd_ref[...].astype(jnp.bfloat16)
    t = lax.dot_general(xb, wx_ref[...].astype(jnp.bfloat16), _NT,
                        preferred_element_type=jnp.float32)
    acc_ref[...] = jnp.full(acc_ref.shape, -jnp.inf, jnp.float32)
    iota = lax.broadcasted_iota(jnp.int32, (n, n), 1)

    def step(_, carry):
        pd = pd_ref[...]
        m = jnp.max(pd, axis=1, keepdims=True)
        first = jnp.min(jnp.where(pd == m, iota, n), axis=1, keepdims=True)
        onehot = (iota == first).astype(jnp.float32)
        rows = lax.dot_general(onehot, x, _NN, precision=_PH)
        d = (rows - x).astype(jnp.bfloat16)
        hk = lax.dot_general(d, wdb, _NT, preferred_element_type=jnp.float32)
        acc_ref[...] = jnp.maximum(acc_ref[...], hk)
        pd_ref[...] = jnp.where(iota == first, -jnp.inf, pd)
        return carry

    lax.fori_loop(0, K, step, 0)
    o_ref[0] = _leaky((acc_ref[...] + t) * sc_ref[...][None, :] + sh_ref[...][None, :])


def _edge_layer(xb, W, gamma, beta):
    B, n, C = xb.shape
    O = W.shape[0]
    Wd = W[:, :C]
    Wx = W[:, C:]
    scale = gamma / np.sqrt(1.0 + 1e-5)
    return pl.pallas_call(
        _edge_body,
        grid=(B,),
        in_specs=[
            pl.BlockSpec((1, n, C), lambda b: (b, 0, 0)),
            pl.BlockSpec((O, C), lambda b: (0, 0)),
            pl.BlockSpec((O, C), lambda b: (0, 0)),
            pl.BlockSpec((O,), lambda b: (0,)),
            pl.BlockSpec((O,), lambda b: (0,)),
        ],
        out_specs=pl.BlockSpec((1, n, O), lambda b: (b, 0, 0)),
        out_shape=jax.ShapeDtypeStruct((B, n, O), jnp.float32),
        scratch_shapes=[
            pltpu.VMEM((n, n), jnp.float32),
            pltpu.VMEM((n, O), jnp.float32),
        ],
    )(xb, Wd, Wx, scale, beta)


def _knn_body(x_ref, wx_ref, o_idx_ref, o_t_ref, pd_ref):
    n = x_ref.shape[1]
    b = pl.program_id(0)
    x = x_ref[0]
    xb = x.astype(jnp.bfloat16)
    g = lax.dot_general(xb, xb, _NT, preferred_element_type=jnp.float32)
    s = jnp.sum(x * x, axis=1)
    pd_ref[...] = 2.0 * g - s[:, None] - s[None, :]
    o_t_ref[0] = lax.dot_general(xb, wx_ref[...].astype(jnp.bfloat16), _NT,
                                 preferred_element_type=jnp.float32)
    iota = lax.broadcasted_iota(jnp.int32, (n, n), 1)
    cols = []
    for _ in range(K):
        pd = pd_ref[...]
        m = jnp.max(pd, axis=1, keepdims=True)
        first = jnp.min(jnp.where(pd == m, iota, n), axis=1, keepdims=True)
        cols.append(first)
        pd_ref[...] = jnp.where(iota == first, -jnp.inf, pd)
    o_idx_ref[0] = jnp.concatenate(cols, axis=1) + b * n


def _sc_gather(x_flat, idx_flat):
    TOT = idx_flat.shape[0]
    C = x_flat.shape[1]
    info = plsc.get_sparse_core_info()
    nw = info.num_cores * info.num_subcores
    ch = 128
    per_w = TOT // nw
    n_ch = per_w // ch
    assert per_w * nw == TOT and n_ch * ch == per_w
    mesh = plsc.VectorSubcoreMesh(core_axis_name="c", subcore_axis_name="s")

    @functools.partial(
        pl.kernel, mesh=mesh,
        compiler_params=pltpu.CompilerParams(use_tc_tiling_on_sc=False),
        out_type=jax.ShapeDtypeStruct((TOT, C), jnp.float32),
        scratch_types=[
            pltpu.VMEM((ch,), jnp.int32),
            pltpu.VMEM((ch, C), jnp.float32),
            pltpu.SemaphoreType.DMA,
        ],
    )
    def gather(x_hbm, idx_hbm, out_hbm, idx_v, rows_v, sem):
        wid = lax.axis_index("s") * info.num_cores + lax.axis_index("c")
        base = wid * per_w

        def body(i, carry):
            off = base + i * ch
            pltpu.sync_copy(idx_hbm.at[pl.ds(off, ch)], idx_v)
            pltpu.async_copy(x_hbm.at[idx_v], rows_v, sem).wait()
            pltpu.sync_copy(rows_v, out_hbm.at[pl.ds(off, ch)])
            return carry

        lax.fori_loop(0, n_ch, body, 0)

    return gather(x_flat, idx_flat)


def _agg_body(feat_ref, x_ref, t_ref, wd_ref, sc_ref, sh_ref, o_ref):
    x = x_ref[0]
    wdb = wd_ref[...].astype(jnp.bfloat16)
    acc = None
    for k in range(K):
        d = (feat_ref[0, :, k, :] - x).astype(jnp.bfloat16)
        hk = lax.dot_general(d, wdb, _NT, preferred_element_type=jnp.float32)
        acc = hk if acc is None else jnp.maximum(acc, hk)
    o_ref[0] = _leaky((acc + t_ref[0]) * sc_ref[...][None, :] + sh_ref[...][None, :])


def _edge_layer_sc(xb, W, gamma, beta):
    B, n, C = xb.shape
    O = W.shape[0]
    Wd = W[:, :C]
    Wx = W[:, C:]
    scale = gamma / np.sqrt(1.0 + 1e-5)
    idx, t = pl.pallas_call(
        _knn_body,
        grid=(B,),
        in_specs=[
            pl.BlockSpec((1, n, C), lambda b: (b, 0, 0)),
            pl.BlockSpec((O, C), lambda b: (0, 0)),
        ],
        out_specs=(
            pl.BlockSpec((1, n, K), lambda b: (b, 0, 0)),
            pl.BlockSpec((1, n, O), lambda b: (b, 0, 0)),
        ),
        out_shape=(
            jax.ShapeDtypeStruct((B, n, K), jnp.int32),
            jax.ShapeDtypeStruct((B, n, O), jnp.float32),
        ),
        scratch_shapes=[pltpu.VMEM((n, n), jnp.float32)],
    )(xb, Wx)
    feat = _sc_gather(xb.reshape(B * n, C), idx.reshape(B * n * K))
    out = pl.pallas_call(
        _agg_body,
        grid=(B,),
        in_specs=[
            pl.BlockSpec((1, n, K, C), lambda b: (b, 0, 0, 0)),
            pl.BlockSpec((1, n, C), lambda b: (b, 0, 0)),
            pl.BlockSpec((1, n, O), lambda b: (b, 0, 0)),
            pl.BlockSpec((O, C), lambda b: (0, 0)),
            pl.BlockSpec((O,), lambda b: (0,)),
            pl.BlockSpec((O,), lambda b: (0,)),
        ],
        out_specs=pl.BlockSpec((1, n, O), lambda b: (b, 0, 0)),
        out_shape=jax.ShapeDtypeStruct((B, n, O), jnp.float32),
    )(feat.reshape(B, n, K, C), xb, t, Wd, scale, beta)
    return out


def _head_body(x1_ref, x2_ref, x3_ref, x4_ref, wa_ref, wb_ref, wc_ref, wd_ref,
               g5_ref, b5_ref, l1a_ref, l1b_ref, g6_ref, b6_ref,
               l2_ref, l2b_ref, g7_ref, b7_ref, l3_ref, l3b_ref, o_ref):
    n = x1_ref.shape[1]
    bf = jnp.bfloat16
    f32 = jnp.float32
    h = (lax.dot_general(x1_ref[0].astype(bf), wa_ref[...].astype(bf), _NT, preferred_element_type=f32)
         + lax.dot_general(x2_ref[0].astype(bf), wb_ref[...].astype(bf), _NT, preferred_element_type=f32)
         + lax.dot_general(x3_ref[0].astype(bf), wc_ref[...].astype(bf), _NT, preferred_element_type=f32)
         + lax.dot_general(x4_ref[0].astype(bf), wd_ref[...].astype(bf), _NT, preferred_element_type=f32))
    h = _leaky(h * g5_ref[...][None, :] + b5_ref[...][None, :])
    p1 = jnp.max(h, axis=0)[None, :]
    p2 = (jnp.sum(h, axis=0) / n)[None, :]
    z = (lax.dot_general(p1.astype(bf), l1a_ref[...].astype(bf), _NT, preferred_element_type=f32)
         + lax.dot_general(p2.astype(bf), l1b_ref[...].astype(bf), _NT, preferred_element_type=f32))
    z = _leaky(z * g6_ref[...][None, :] + b6_ref[...][None, :])
    z = lax.dot_general(z.astype(bf), l2_ref[...].astype(bf), _NT, preferred_element_type=f32) + l2b_ref[...][None, :]
    z = _leaky(z * g7_ref[...][None, :] + b7_ref[...][None, :])
    z = lax.dot_general(z.astype(bf), l3_ref[...].astype(bf), _NT, preferred_element_type=f32) + l3b_ref[...][None, :]
    o_ref[0] = z


def _head(x1, x2, x3, x4, W5, g5, b5, L1W, g6, b6, L2W, L2b, g7, b7, L3W, L3b):
    B, n, _ = x1.shape
    emb = W5.shape[0]
    c1, c2, c3, c4 = x1.shape[2], x2.shape[2], x3.shape[2], x4.shape[2]
    Wa = W5[:, :c1]
    Wb = W5[:, c1:c1 + c2]
    Wc = W5[:, c1 + c2:c1 + c2 + c3]
    Wd = W5[:, c1 + c2 + c3:]
    L1a = L1W[:, :emb]
    L1b = L1W[:, emb:]
    full = lambda shape: pl.BlockSpec(shape, lambda b: (0,) * len(shape))
    return pl.pallas_call(
        _head_body,
        grid=(B,),
        in_specs=[
            pl.BlockSpec((1, n, c1), lambda b: (b, 0, 0)),
            pl.BlockSpec((1, n, c2), lambda b: (b, 0, 0)),
            pl.BlockSpec((1, n, c3), lambda b: (b, 0, 0)),
            pl.BlockSpec((1, n, c4), lambda b: (b, 0, 0)),
            full(Wa.shape), full(Wb.shape), full(Wc.shape), full(Wd.shape),
            full(g5.shape), full(b5.shape),
            full(L1a.shape), full(L1b.shape), full(g6.shape), full(b6.shape),
            full(L2W.shape), full(L2b.shape), full(g7.shape), full(b7.shape),
            full(L3W.shape), full(L3b.shape),
        ],
        out_specs=pl.BlockSpec((1, 1, 40), lambda b: (b, 0, 0)),
        out_shape=jax.ShapeDtypeStruct((B, 1, 40), jnp.float32),
    )(x1, x2, x3, x4, Wa, Wb, Wc, Wd, g5 / np.sqrt(1.0 + 1e-5), b5,
      L1a, L1b, g6 / np.sqrt(1.0 + 1e-5), b6, L2W, L2b, g7 / np.sqrt(1.0 + 1e-5), b7, L3W, L3b)


def kernel(x, normal, W1, g1, b1, W2, g2, b2, W3, g3, b3, W4, g4, b4, W5, g5, b5, L1W, g6, b6, L2W, L2b, g7, b7, L3W, L3b):
    del normal
    B = x.shape[0]
    x0 = jnp.concatenate(
        [x.astype(jnp.float32), jnp.zeros((B, N, 13), jnp.float32)], axis=2)
    W1p = jnp.concatenate(
        [W1[:, :3], jnp.zeros((64, 13), jnp.float32),
         W1[:, 3:], jnp.zeros((64, 13), jnp.float32)], axis=1)
    x1 = _edge_layer_sc(x0, W1p, g1, b1)
    x2 = _edge_layer_sc(x1, W2, g2, b2)
    x3 = _edge_layer_sc(x2, W3, g3, b3)
    x4 = _edge_layer_sc(x3, W4, g4, b4)
    out = _head(x1, x2, x3, x4, W5, g5, b5, L1W, g6, b6,
                L2W, L2b, g7, b7, L3W, L3b)
    return out.reshape(B, 40)
